# baseline (speedup 1.0000x reference)
import functools

import jax
import jax.numpy as jnp
from jax import lax
from jax.experimental import pallas as pl
from jax.experimental.pallas import tpu as pltpu
from jax.experimental.pallas import tpu_sc as plsc

N = 10000
D = 128
E = 320000
NC = 2
NS = 16
NW = NC * NS
EPW = E // NW
B = 40
STEPS = EPW // B
NP = 10240
RPS = NP // NS

def _deg_body(dst_hbm, deg_out, dst_v, hist_v, sem):
    c = lax.axis_index("c")
    s = lax.axis_index("s")
    w = s * NC + c
    pltpu.sync_copy(dst_hbm.at[pl.ds(w * EPW, EPW)], dst_v)

    def zero(k, carry):
        hist_v[pl.ds(k * 16, 16)] = jnp.zeros((16,), jnp.float32)
        return carry

    lax.fori_loop(0, NP // 16, zero, 0)
    one16 = jnp.ones((16,), jnp.float32)

    def step(k, carry):
        plsc.addupdate_scatter(hist_v, [dst_v[pl.ds(k * 16, 16)]], one16)
        return carry

    lax.fori_loop(0, EPW // 16, step, 0)
    pltpu.sync_copy(hist_v, deg_out.at[c, s])


CS = 50
CHUNKS = STEPS // CS
NBUF = 5
LOOK = 2


def _agg_body(x_hbm, src_hbm, dst_hbm, zrow_hbm, dep_hbm, agg_out,
              srcb, dstb, r0, r1, r2, r3, r4,
              g0, g1, g2, g3, g4, s0, s1, s2, s3, s4, agg_sh):
    del dep_hbm
    rows = (r0, r1, r2, r3, r4)
    gsem = (g0, g1, g2, g3, g4)
    ssem = (s0, s1, s2, s3, s4)
    c = lax.axis_index("c")
    s = lax.axis_index("s")
    w = s * NC + c
    pltpu.sync_copy(zrow_hbm.at[pl.ds(s * RPS, RPS)],
                    agg_sh.at[pl.ds(s * RPS, RPS)])
    plsc.subcore_barrier()

    def chunk(cc, carry):
        pltpu.sync_copy(src_hbm.at[pl.ds(w * EPW + cc * CS * B, CS * B)],
                        srcb)
        pltpu.sync_copy(dst_hbm.at[w, cc], dstb)

        def gather(j, b):
            pltpu.async_copy(x_hbm.at[srcb.at[pl.ds(j * B, B)]], rows[b],
                             gsem[b])

        def gather_wait(j, b):
            pltpu.make_async_copy(x_hbm.at[srcb.at[pl.ds(j * B, B)]],
                                  rows[b], gsem[b]).wait()

        def scatter(j, b):
            pltpu.async_copy(rows[b], agg_sh.at[dstb.at[j]], ssem[b],
                             add=True)

        def scatter_wait(j, b):
            pltpu.make_async_copy(rows[b], agg_sh.at[dstb.at[j]],
                                  ssem[b]).wait()

        for j in range(LOOK):
            gather(j, j)

        def group(k, carry2):
            for b in range(NBUF):
                j = k * NBUF + b
                bp = (b + LOOK) % NBUF
                @pl.when(j >= NBUF - LOOK)
                def _():
                    scatter_wait(j - (NBUF - LOOK), bp)

                @pl.when(j + LOOK < CS)
                def _():
                    gather(j + LOOK, bp)

                gather_wait(j, b)
                scatter(j, b)
            return carry2

        lax.fori_loop(0, CS // NBUF, group, 0)
        for j in range(CS - (NBUF - LOOK), CS):
            scatter_wait(j, j % NBUF)
        return carry

    lax.fori_loop(0, CHUNKS, chunk, 0)
    plsc.subcore_barrier()
    pltpu.sync_copy(agg_sh.at[pl.ds(s * RPS, RPS)],
                    agg_out.at[c, pl.ds(s * RPS, RPS)])


@functools.cache
def _build_agg(with_deg):
    mesh = plsc.VectorSubcoreMesh(core_axis_name="c", subcore_axis_name="s")
    if with_deg:
        return pl.kernel(
            _deg_body,
            out_type=[jax.ShapeDtypeStruct((NC, NS, NP), jnp.float32)],
            mesh=mesh,
            compiler_params=pltpu.CompilerParams(needs_layout_passes=False),
            scratch_types=[
                pltpu.VMEM((EPW,), jnp.int32),
                pltpu.VMEM((NP,), jnp.float32),
                pltpu.SemaphoreType.DMA,
            ],
        )
    return pl.kernel(
        _agg_body,
        out_type=[jax.ShapeDtypeStruct((NC, NP, D), jnp.float32)],
        mesh=mesh,
        scratch_types=(
            [pltpu.VMEM((CS * B,), jnp.int32),
             pltpu.VMEM((CS, B), jnp.int32)]
            + [pltpu.VMEM((B, D), jnp.float32)] * NBUF
            + [pltpu.SemaphoreType.DMA] * (2 * NBUF)
            + [pltpu.VMEM_SHARED((NP, D), jnp.float32)]
        ),
    )

MB = 1000


def _dense1_body(x, p, dp, w1, b1, o):
    deg = jnp.maximum(jnp.sum(dp[...], axis=1), 1.0).reshape(MB, 1)
    n1 = (p[0] + p[1]) / deg
    h = (jnp.dot(x[...], w1[0:D, :], preferred_element_type=jnp.float32)
         + jnp.dot(n1, w1[D:2 * D, :], preferred_element_type=jnp.float32)
         + b1[...])
    o[...] = jnp.maximum(h, 0.0)


def _dense2_body(h1, p, dp, w2, b2, wo, bo, o):
    deg = jnp.maximum(jnp.sum(dp[...], axis=1), 1.0).reshape(MB, 1)
    n2 = (p[0] + p[1]) / deg
    h = (jnp.dot(h1[...], w2[0:D, :], preferred_element_type=jnp.float32)
         + jnp.dot(n2, w2[D:2 * D, :], preferred_element_type=jnp.float32)
         + b2[...])
    h = jnp.maximum(h, 0.0)
    logits = jnp.dot(h, wo[...], preferred_element_type=jnp.float32) + bo[...]
    o[...] = jax.nn.sigmoid(logits)


_row = lambda i: (i, 0)
_rep = lambda i: (0, 0)

_dense1 = pl.pallas_call(
    _dense1_body,
    grid=(N // MB,),
    in_specs=[
        pl.BlockSpec((MB, D), _row),
        pl.BlockSpec((NC, MB, D), lambda i: (0, i, 0)),
        pl.BlockSpec((MB, NW), _row),
        pl.BlockSpec((2 * D, D), _rep),
        pl.BlockSpec((1, D), _rep),
    ],
    out_specs=pl.BlockSpec((MB, D), _row),
    out_shape=jax.ShapeDtypeStruct((N, D), jnp.float32),
)

_dense2 = pl.pallas_call(
    _dense2_body,
    grid=(N // MB,),
    in_specs=[
        pl.BlockSpec((MB, D), _row),
        pl.BlockSpec((NC, MB, D), lambda i: (0, i, 0)),
        pl.BlockSpec((MB, NW), _row),
        pl.BlockSpec((2 * D, D), _rep),
        pl.BlockSpec((1, D), _rep),
        pl.BlockSpec((D, 1), _rep),
        pl.BlockSpec((1, 1), _rep),
    ],
    out_specs=pl.BlockSpec((MB, 1), _row),
    out_shape=jax.ShapeDtypeStruct((N, 1), jnp.float32),
)


def kernel(x, edge_index, W1, b1, W2, b2, Wo, bo):
    src = edge_index[0].astype(jnp.int32)
    dstf = edge_index[1].astype(jnp.int32)
    dst = dstf.reshape(NW, CHUNKS, CS, B)
    zrow = jnp.zeros((NP, D), jnp.float32)

    deg = _build_agg(True)(dstf)
    if isinstance(deg, (list, tuple)):
        deg = deg[0]
    degp = deg.reshape(NW, NP)[:, :N].T
    dep = degp[:8, :NW]
    agg1 = _build_agg(False)(x, src, dst, zrow, dep)
    if isinstance(agg1, (list, tuple)):
        agg1 = agg1[0]
    h1 = _dense1(x, agg1, degp, W1, b1.reshape(1, D))
    agg2 = _build_agg(False)(h1, src, dst, zrow, dep)
    if isinstance(agg2, (list, tuple)):
        agg2 = agg2[0]
    out = _dense2(h1, agg2, degp, W2, b2.reshape(1, D), Wo,
                  bo.reshape(1, 1))
    return out


# speedup vs baseline: 1.0602x; 1.0602x over previous
import functools

import jax
import jax.numpy as jnp
from jax import lax
from jax.experimental import pallas as pl
from jax.experimental.pallas import tpu as pltpu
from jax.experimental.pallas import tpu_sc as plsc

N = 10000
D = 128
E = 320000
NC = 2
NS = 16
NW = NC * NS
EPW = E // NW
B = 40
STEPS = EPW // B
NP = 10240
RPS = NP // NS

def _deg_body(dst_hbm, deg_out, dst_v, hist_v, sem):
    c = lax.axis_index("c")
    s = lax.axis_index("s")
    w = s * NC + c
    pltpu.sync_copy(dst_hbm.at[pl.ds(w * EPW, EPW)], dst_v)

    def zero(k, carry):
        hist_v[pl.ds(k * 16, 16)] = jnp.zeros((16,), jnp.float32)
        return carry

    lax.fori_loop(0, NP // 16, zero, 0)
    one16 = jnp.ones((16,), jnp.float32)

    def step(k, carry):
        plsc.addupdate_scatter(hist_v, [dst_v[pl.ds(k * 16, 16)]], one16)
        return carry

    lax.fori_loop(0, EPW // 16, step, 0)
    pltpu.sync_copy(hist_v, deg_out.at[c, s])


CS = 50
CHUNKS = STEPS // CS
NBUF = 5
LOOK = 3


def _agg_body(x_hbm, src_hbm, dst_hbm, zrow_hbm, dep_hbm, agg_out,
              srcb, dstb, r0, r1, r2, r3, r4,
              g0, g1, g2, g3, g4, s0, s1, s2, s3, s4, agg_sh):
    del dep_hbm
    rows = (r0, r1, r2, r3, r4)
    gsem = (g0, g1, g2, g3, g4)
    ssem = (s0, s1, s2, s3, s4)
    c = lax.axis_index("c")
    s = lax.axis_index("s")
    w = s * NC + c
    pltpu.sync_copy(zrow_hbm.at[pl.ds(s * RPS, RPS)],
                    agg_sh.at[pl.ds(s * RPS, RPS)])
    plsc.subcore_barrier()

    def chunk(cc, carry):
        pltpu.sync_copy(src_hbm.at[pl.ds(w * EPW + cc * CS * B, CS * B)],
                        srcb)
        pltpu.sync_copy(dst_hbm.at[w, cc], dstb)

        def gather(j, b):
            pltpu.async_copy(x_hbm.at[srcb.at[pl.ds(j * B, B)]], rows[b],
                             gsem[b])

        def gather_wait(j, b):
            pltpu.make_async_copy(x_hbm.at[srcb.at[pl.ds(j * B, B)]],
                                  rows[b], gsem[b]).wait()

        def scatter(j, b):
            pltpu.async_copy(rows[b], agg_sh.at[dstb.at[j]], ssem[b],
                             add=True)

        def scatter_wait(j, b):
            pltpu.make_async_copy(rows[b], agg_sh.at[dstb.at[j]],
                                  ssem[b]).wait()

        for j in range(LOOK):
            gather(j, j)

        def group(k, carry2):
            for b in range(NBUF):
                j = k * NBUF + b
                bp = (b + LOOK) % NBUF
                @pl.when(j >= NBUF - LOOK)
                def _():
                    scatter_wait(j - (NBUF - LOOK), bp)

                @pl.when(j + LOOK < CS)
                def _():
                    gather(j + LOOK, bp)

                gather_wait(j, b)
                scatter(j, b)
            return carry2

        lax.fori_loop(0, CS // NBUF, group, 0)
        for j in range(CS - (NBUF - LOOK), CS):
            scatter_wait(j, j % NBUF)
        return carry

    lax.fori_loop(0, CHUNKS, chunk, 0)
    plsc.subcore_barrier()
    pltpu.sync_copy(agg_sh.at[pl.ds(s * RPS, RPS)],
                    agg_out.at[c, pl.ds(s * RPS, RPS)])


@functools.cache
def _build_agg(with_deg):
    mesh = plsc.VectorSubcoreMesh(core_axis_name="c", subcore_axis_name="s")
    if with_deg:
        return pl.kernel(
            _deg_body,
            out_type=[jax.ShapeDtypeStruct((NC, NS, NP), jnp.float32)],
            mesh=mesh,
            compiler_params=pltpu.CompilerParams(needs_layout_passes=False),
            scratch_types=[
                pltpu.VMEM((EPW,), jnp.int32),
                pltpu.VMEM((NP,), jnp.float32),
                pltpu.SemaphoreType.DMA,
            ],
        )
    return pl.kernel(
        _agg_body,
        out_type=[jax.ShapeDtypeStruct((NC, NP, D), jnp.float32)],
        mesh=mesh,
        scratch_types=(
            [pltpu.VMEM((CS * B,), jnp.int32),
             pltpu.VMEM((CS, B), jnp.int32)]
            + [pltpu.VMEM((B, D), jnp.float32)] * NBUF
            + [pltpu.SemaphoreType.DMA] * (2 * NBUF)
            + [pltpu.VMEM_SHARED((NP, D), jnp.float32)]
        ),
    )

MB = 1000


def _dense1_body(x, p, dp, w1, b1, o):
    deg = jnp.maximum(jnp.sum(dp[...], axis=1), 1.0).reshape(MB, 1)
    n1 = (p[0] + p[1]) / deg
    h = (jnp.dot(x[...], w1[0:D, :], preferred_element_type=jnp.float32)
         + jnp.dot(n1, w1[D:2 * D, :], preferred_element_type=jnp.float32)
         + b1[...])
    o[...] = jnp.maximum(h, 0.0)


def _dense2_body(h1, p, dp, w2, b2, wo, bo, o):
    deg = jnp.maximum(jnp.sum(dp[...], axis=1), 1.0).reshape(MB, 1)
    n2 = (p[0] + p[1]) / deg
    h = (jnp.dot(h1[...], w2[0:D, :], preferred_element_type=jnp.float32)
         + jnp.dot(n2, w2[D:2 * D, :], preferred_element_type=jnp.float32)
         + b2[...])
    h = jnp.maximum(h, 0.0)
    logits = jnp.dot(h, wo[...], preferred_element_type=jnp.float32) + bo[...]
    o[...] = jax.nn.sigmoid(logits)


_row = lambda i: (i, 0)
_rep = lambda i: (0, 0)

_dense1 = pl.pallas_call(
    _dense1_body,
    grid=(N // MB,),
    in_specs=[
        pl.BlockSpec((MB, D), _row),
        pl.BlockSpec((NC, MB, D), lambda i: (0, i, 0)),
        pl.BlockSpec((MB, NW), _row),
        pl.BlockSpec((2 * D, D), _rep),
        pl.BlockSpec((1, D), _rep),
    ],
    out_specs=pl.BlockSpec((MB, D), _row),
    out_shape=jax.ShapeDtypeStruct((N, D), jnp.float32),
)

_dense2 = pl.pallas_call(
    _dense2_body,
    grid=(N // MB,),
    in_specs=[
        pl.BlockSpec((MB, D), _row),
        pl.BlockSpec((NC, MB, D), lambda i: (0, i, 0)),
        pl.BlockSpec((MB, NW), _row),
        pl.BlockSpec((2 * D, D), _rep),
        pl.BlockSpec((1, D), _rep),
        pl.BlockSpec((D, 1), _rep),
        pl.BlockSpec((1, 1), _rep),
    ],
    out_specs=pl.BlockSpec((MB, 1), _row),
    out_shape=jax.ShapeDtypeStruct((N, 1), jnp.float32),
)


def kernel(x, edge_index, W1, b1, W2, b2, Wo, bo):
    src = edge_index[0].astype(jnp.int32)
    dstf = edge_index[1].astype(jnp.int32)
    dst = dstf.reshape(NW, CHUNKS, CS, B)
    zrow = jnp.zeros((NP, D), jnp.float32)

    deg = _build_agg(True)(dstf)
    if isinstance(deg, (list, tuple)):
        deg = deg[0]
    degp = deg.reshape(NW, NP)[:, :N].T
    dep = degp[:8, :NW]
    agg1 = _build_agg(False)(x, src, dst, zrow, dep)
    if isinstance(agg1, (list, tuple)):
        agg1 = agg1[0]
    h1 = _dense1(x, agg1, degp, W1, b1.reshape(1, D))
    agg2 = _build_agg(False)(h1, src, dst, zrow, dep)
    if isinstance(agg2, (list, tuple)):
        agg2 = agg2[0]
    out = _dense2(h1, agg2, degp, W2, b2.reshape(1, D), Wo,
                  bo.reshape(1, 1))
    return out
